# baseline (speedup 1.0000x reference)
import functools

import jax
import jax.numpy as jnp
from jax import lax
from jax.experimental import pallas as pl
from jax.experimental.pallas import tpu as pltpu
from jax.experimental.pallas import tpu_sc as plsc

NNODE = 10000
NEDGE = 160000
FD = 16
HMLP = 32

NC, NS = 2, 16
NW = NC * NS
CH = 128
GRP = 8
NCH = 40
PERW = NCH * CH
EPAD = NW * PERW
RPT = 626
NPAD = RPT * NS
SENT = NNODE
TE = 3200
M8 = TE // 8


def _sc_mesh():
    return plsc.VectorSubcoreMesh(
        core_axis_name="c", subcore_axis_name="s",
        num_cores=NC, num_subcores=NS)


GN0 = 24
GN1 = 56
GNMX = max(GN0, GN1)
NCHT = EPAD // CH
NCHP = NCHT + GNMX - min(GN0, GN1)


@functools.cache
def _make_sc_gather():
    def body(x_hbm, idx_hbm, out_hbm, idx_v, rows_v, sem):
        cid = lax.axis_index("c")
        sid = lax.axis_index("s")
        start = jnp.where(cid == 0, sid * GN0, NS * GN0 + sid * GN1)
        ngrp = jnp.where(cid == 0, GN0 // GRP, GN1 // GRP)
        pltpu.sync_copy(idx_hbm.at[pl.ds(start, GNMX)], idx_v)
        for g in range(GNMX // GRP):
            @pl.when(g < ngrp)
            def _():
                cps = []
                for b in range(GRP):
                    j = g * GRP + b
                    cps.append(pltpu.async_copy(
                        x_hbm.at[idx_v.at[j]],
                        rows_v.at[pl.ds(b * CH, CH)], sem))
                for cp in cps:
                    cp.wait()
                pltpu.sync_copy(
                    rows_v,
                    out_hbm.at[pl.ds((start + g * GRP) * CH, GRP * CH)])

    return pl.kernel(
        body,
        out_type=jax.ShapeDtypeStruct((EPAD, FD), jnp.float32),
        mesh=_sc_mesh(),
        compiler_params=pltpu.CompilerParams(use_tc_tiling_on_sc=False),
        scratch_types=[
            pltpu.VMEM((GNMX, CH), jnp.int32),
            pltpu.VMEM((GRP * CH, FD), jnp.float32),
            pltpu.SemaphoreType.DMA,
        ])


def _sc_gather(x, idx2):
    return _make_sc_gather()(x, idx2)


@functools.cache
def _make_sc_scatter(do_deg):
    out_type = [jax.ShapeDtypeStruct((NC, NPAD, FD), jnp.float32)]
    scratch = [
        pltpu.VMEM((NCH, CH), jnp.int32),
        pltpu.VMEM((GRP * CH, FD), jnp.float32),
        pltpu.VMEM_SHARED((NPAD, FD), jnp.float32),
        pltpu.SemaphoreType.DMA,
    ]
    if do_deg:
        out_type.append(jax.ShapeDtypeStruct((NC, NPAD, FD), jnp.float32))
        scratch.append(pltpu.VMEM((CH, FD), jnp.float32))
        scratch.append(pltpu.VMEM_SHARED((NPAD, FD), jnp.float32))

    def body(*refs):
        if do_deg:
            (msg_hbm, idx_hbm, zeros_hbm, ones_hbm, out_hbm, deg_hbm,
             idx_v, msg_v, acc_sh, sem, ones_v, deg_sh) = refs
        else:
            (msg_hbm, idx_hbm, zeros_hbm, out_hbm,
             idx_v, msg_v, acc_sh, sem) = refs
        cid = lax.axis_index("c")
        sid = lax.axis_index("s")
        wid = sid * NC + cid
        r0 = sid * RPT
        pltpu.sync_copy(zeros_hbm.at[pl.ds(r0, RPT)],
                        acc_sh.at[pl.ds(r0, RPT)])
        if do_deg:
            pltpu.sync_copy(zeros_hbm.at[pl.ds(r0, RPT)],
                            deg_sh.at[pl.ds(r0, RPT)])
            pltpu.sync_copy(ones_hbm, ones_v)
        pltpu.sync_copy(idx_hbm.at[wid], idx_v)
        plsc.subcore_barrier()
        for g in range(NCH // GRP):
            pltpu.sync_copy(
                msg_hbm.at[pl.ds(wid * PERW + g * GRP * CH, GRP * CH)], msg_v)
            for b in range(GRP):
                j = g * GRP + b
                pltpu.sync_copy(msg_v.at[pl.ds(b * CH, CH)],
                                acc_sh.at[idx_v.at[j]], add=True)
                if do_deg:
                    pltpu.sync_copy(ones_v, deg_sh.at[idx_v.at[j]], add=True)
        plsc.subcore_barrier()
        pltpu.sync_copy(acc_sh.at[pl.ds(r0, RPT)],
                        out_hbm.at[cid].at[pl.ds(r0, RPT)])
        if do_deg:
            pltpu.sync_copy(deg_sh.at[pl.ds(r0, RPT)],
                            deg_hbm.at[cid].at[pl.ds(r0, RPT)])

    if not do_deg:
        out_type = out_type[0]
    return pl.kernel(
        body, out_type=out_type, mesh=_sc_mesh(),
        compiler_params=pltpu.CompilerParams(use_tc_tiling_on_sc=False),
        scratch_types=scratch)


def _sc_scatter_deg(msg, idx3, zeros_np, ones_ch):
    return _make_sc_scatter(True)(msg, idx3, zeros_np, ones_ch)


def _sc_scatter(msg, idx3, zeros_np):
    return _make_sc_scatter(False)(msg, idx3, zeros_np)


def _msg_body(ea_ref, xs_ref, w1_ref, b1_ref, w2_ref, b2_ref, q_ref, s_ref,
              out_ref):
    f32 = jnp.float32
    ea40 = ea_ref[...]
    ea = jnp.concatenate(
        [ea40[:, e * 5:(e + 1) * 5] for e in range(8)], axis=0)
    xs8 = xs_ref[...]
    xs = jnp.concatenate(
        [xs8[:, e * FD:(e + 1) * FD] for e in range(8)], axis=0)
    h = jnp.maximum(
        jnp.dot(ea, w1_ref[...], preferred_element_type=f32)
        + b1_ref[...], 0.0)
    p = jnp.dot(h, w2_ref[...], preferred_element_type=f32) + b2_ref[...]
    xr = jnp.dot(xs, q_ref[...], preferred_element_type=f32)
    msg = jnp.dot(p * xr, s_ref[...], preferred_element_type=f32)
    out_ref[...] = jnp.concatenate(
        [msg[e * M8:(e + 1) * M8, :] for e in range(8)], axis=1)


def _msg_call(ea40, xs128, w1, b1r, w2, b2r, qmat, smat):
    c0 = lambda i: (0, 0)
    return pl.pallas_call(
        _msg_body,
        grid=(NEDGE // TE,),
        in_specs=[
            pl.BlockSpec((M8, 40), lambda i: (i, 0)),
            pl.BlockSpec((M8, 128), lambda i: (i, 0)),
            pl.BlockSpec((5, HMLP), c0),
            pl.BlockSpec((1, HMLP), c0),
            pl.BlockSpec((HMLP, FD * FD), c0),
            pl.BlockSpec((1, FD * FD), c0),
            pl.BlockSpec((FD, FD * FD), c0),
            pl.BlockSpec((FD * FD, FD), c0),
        ],
        out_specs=pl.BlockSpec((M8, 128), lambda i: (i, 0)),
        out_shape=jax.ShapeDtypeStruct((EPAD // 8, 128), jnp.float32),
    )(ea40, xs128, w1, b1r, w2, b2r, qmat, smat)


def _bn_relu(y, gamma, beta):
    m = jnp.mean(y, axis=0, keepdims=True)
    v = jnp.mean((y - m) * (y - m), axis=0, keepdims=True)
    return jnp.maximum((y - m) / jnp.sqrt(v + 1e-5) * gamma + beta, 0.0)


def _sum_parts(p):
    return p[0, :NNODE, :] + p[1, :NNODE, :]


def _combine1_body(p_ref, dp_ref, x_ref, root_ref, bias_ref, gamma_ref,
                   beta_ref, x2_ref, deg_ref):
    agg = _sum_parts(p_ref[...])
    deg = _sum_parts(dp_ref[...])
    y = (agg / jnp.maximum(deg, 1.0)
         + jnp.dot(x_ref[...], root_ref[...],
                   preferred_element_type=jnp.float32)
         + bias_ref[...])
    x2_ref[...] = _bn_relu(y, gamma_ref[...], beta_ref[...])
    deg_ref[...] = deg


def _combine1(parts, degp, x, root, bias, gamma, beta):
    return pl.pallas_call(
        _combine1_body,
        out_shape=[jax.ShapeDtypeStruct((NNODE, FD), jnp.float32),
                   jax.ShapeDtypeStruct((NNODE, FD), jnp.float32)],
    )(parts, degp, x, root, bias, gamma, beta)


def _combine2_body(p_ref, deg_ref, x_ref, root_ref, bias_ref, gamma_ref,
                   beta_ref, batch_ref, ga_ref, wh_ref, wg_ref, bfc_ref,
                   out_ref):
    f32 = jnp.float32
    agg = _sum_parts(p_ref[...])
    y = (agg / jnp.maximum(deg_ref[...], 1.0)
         + jnp.dot(x_ref[...], root_ref[...], preferred_element_type=f32)
         + bias_ref[...])
    h2 = _bn_relu(y, gamma_ref[...], beta_ref[...])
    gi = lax.broadcasted_iota(jnp.int32, (8, NNODE), 0)
    oh = (jnp.broadcast_to(batch_ref[...], (8, NNODE)) == gi).astype(f32)
    cnt = jnp.sum(oh, axis=1, keepdims=True)
    pooled = (jnp.dot(oh, h2, preferred_element_type=f32)
              / jnp.maximum(cnt, 1.0))
    out_ref[...] = (jnp.dot(pooled, wh_ref[...], preferred_element_type=f32)
                    + jnp.dot(ga_ref[...], wg_ref[...],
                              preferred_element_type=f32)
                    + bfc_ref[...])


def _combine2(parts, deg16, x2, root, bias, gamma, beta, batch2d, ga,
              wh, wg, bfc2d):
    return pl.pallas_call(
        _combine2_body,
        out_shape=jax.ShapeDtypeStruct((8, 128), jnp.float32),
    )(parts, deg16, x2, root, bias, gamma, beta, batch2d, ga, wh, wg, bfc2d)


def kernel(x, edge_index, edge_attr, batch, graph_attr, W1, b1, W2, b2,
           root1, bias1, gamma1, beta1, W3, b3, W4, b4, root2, bias2,
           gamma2, beta2, Wfc, bfc):
    f32 = jnp.float32
    src = edge_index[0].astype(jnp.int32)
    dst = edge_index[1].astype(jnp.int32)
    pad = EPAD - NEDGE
    src_p = jnp.concatenate(
        [src, jnp.zeros((pad + (NCHP - NCHT) * CH,), jnp.int32)]
    ).reshape(NCHP, CH)
    dst_p = jnp.concatenate(
        [dst, jnp.full((pad,), SENT, jnp.int32)]).reshape(NW, NCH, CH)
    ea40 = edge_attr.reshape(NEDGE // 8, 40)
    zeros_np = jnp.zeros((NPAD, FD), f32)
    ones_ch = jnp.ones((CH, FD), f32)

    qmat = jnp.repeat(jnp.eye(FD, dtype=f32), FD, axis=1)
    smat = jnp.tile(jnp.eye(FD, dtype=f32), (FD, 1))
    xs1 = _sc_gather(x, src_p)
    msg1 = _msg_call(ea40, xs1.reshape(EPAD // 8, 128), W1,
                     b1.reshape(1, HMLP), W2, b2.reshape(1, FD * FD),
                     qmat, smat)
    agg1, degp = _sc_scatter_deg(msg1.reshape(EPAD, FD), dst_p, zeros_np,
                                 ones_ch)
    x2, deg16 = _combine1(agg1, degp, x, root1, bias1.reshape(1, FD),
                          gamma1.reshape(1, FD), beta1.reshape(1, FD))

    xs2 = _sc_gather(x2, src_p)
    msg2 = _msg_call(ea40, xs2.reshape(EPAD // 8, 128), W3,
                     b3.reshape(1, HMLP), W4, b4.reshape(1, FD * FD),
                     qmat, smat)
    agg2 = _sc_scatter(msg2.reshape(EPAD, FD), dst_p, zeros_np)

    wh = jnp.zeros((FD, 128), f32).at[:, :3].set(Wfc[:FD])
    wg = jnp.zeros((16, 128), f32).at[:10, :3].set(Wfc[FD:])
    ga_p = jnp.zeros((8, 16), f32).at[:, :10].set(graph_attr)
    bfc2d = jnp.zeros((1, 128), f32).at[0, :3].set(bfc)
    out = _combine2(agg2, deg16, x2, root2, bias2.reshape(1, FD),
                    gamma2.reshape(1, FD), beta2.reshape(1, FD),
                    batch.astype(jnp.int32).reshape(1, NNODE), ga_p,
                    wh, wg, bfc2d)
    return out[:, :3]


# speedup vs baseline: 1.0183x; 1.0183x over previous
import functools

import jax
import jax.numpy as jnp
from jax import lax
from jax.experimental import pallas as pl
from jax.experimental.pallas import tpu as pltpu
from jax.experimental.pallas import tpu_sc as plsc

NNODE = 10000
NEDGE = 160000
FD = 16
HMLP = 32

NC, NS = 2, 16
NW = NC * NS
CH = 128
GRP = 8
NCH = 40
PERW = NCH * CH
EPAD = NW * PERW
RPT = 626
NPAD = RPT * NS
SENT = NNODE
TE = 3200
M8 = TE // 8


def _sc_mesh():
    return plsc.VectorSubcoreMesh(
        core_axis_name="c", subcore_axis_name="s",
        num_cores=NC, num_subcores=NS)


GN0 = 56
GN1 = 24
GNMX = max(GN0, GN1)
NCHT = EPAD // CH
NCHP = NCHT + GNMX - min(GN0, GN1)


@functools.cache
def _make_sc_gather():
    def body(x_hbm, idx_hbm, out_hbm, idx_v, rows_v, sem):
        cid = lax.axis_index("c")
        sid = lax.axis_index("s")
        start = jnp.where(cid == 0, sid * GN0, NS * GN0 + sid * GN1)
        ngrp = jnp.where(cid == 0, GN0 // GRP, GN1 // GRP)
        pltpu.sync_copy(idx_hbm.at[pl.ds(start, GNMX)], idx_v)
        for g in range(GNMX // GRP):
            @pl.when(g < ngrp)
            def _():
                cps = []
                for b in range(GRP):
                    j = g * GRP + b
                    cps.append(pltpu.async_copy(
                        x_hbm.at[idx_v.at[j]],
                        rows_v.at[pl.ds(b * CH, CH)], sem))
                for cp in cps:
                    cp.wait()
                pltpu.sync_copy(
                    rows_v,
                    out_hbm.at[pl.ds((start + g * GRP) * CH, GRP * CH)])

    return pl.kernel(
        body,
        out_type=jax.ShapeDtypeStruct((EPAD, FD), jnp.float32),
        mesh=_sc_mesh(),
        compiler_params=pltpu.CompilerParams(use_tc_tiling_on_sc=False),
        scratch_types=[
            pltpu.VMEM((GNMX, CH), jnp.int32),
            pltpu.VMEM((GRP * CH, FD), jnp.float32),
            pltpu.SemaphoreType.DMA,
        ])


def _sc_gather(x, idx2):
    return _make_sc_gather()(x, idx2)


@functools.cache
def _make_sc_scatter(do_deg):
    out_type = [jax.ShapeDtypeStruct((NC, NPAD, FD), jnp.float32)]
    scratch = [
        pltpu.VMEM((NCH, CH), jnp.int32),
        pltpu.VMEM((GRP * CH, FD), jnp.float32),
        pltpu.VMEM_SHARED((NPAD, FD), jnp.float32),
        pltpu.SemaphoreType.DMA,
    ]
    if do_deg:
        out_type.append(jax.ShapeDtypeStruct((NC, NPAD, FD), jnp.float32))
        scratch.append(pltpu.VMEM((CH, FD), jnp.float32))
        scratch.append(pltpu.VMEM_SHARED((NPAD, FD), jnp.float32))

    def body(*refs):
        if do_deg:
            (msg_hbm, idx_hbm, zeros_hbm, ones_hbm, out_hbm, deg_hbm,
             idx_v, msg_v, acc_sh, sem, ones_v, deg_sh) = refs
        else:
            (msg_hbm, idx_hbm, zeros_hbm, out_hbm,
             idx_v, msg_v, acc_sh, sem) = refs
        cid = lax.axis_index("c")
        sid = lax.axis_index("s")
        wid = sid * NC + cid
        r0 = sid * RPT
        pltpu.sync_copy(zeros_hbm.at[pl.ds(r0, RPT)],
                        acc_sh.at[pl.ds(r0, RPT)])
        if do_deg:
            pltpu.sync_copy(zeros_hbm.at[pl.ds(r0, RPT)],
                            deg_sh.at[pl.ds(r0, RPT)])
            pltpu.sync_copy(ones_hbm, ones_v)
        pltpu.sync_copy(idx_hbm.at[wid], idx_v)
        plsc.subcore_barrier()
        for g in range(NCH // GRP):
            pltpu.sync_copy(
                msg_hbm.at[pl.ds(wid * PERW + g * GRP * CH, GRP * CH)], msg_v)
            for b in range(GRP):
                j = g * GRP + b
                pltpu.sync_copy(msg_v.at[pl.ds(b * CH, CH)],
                                acc_sh.at[idx_v.at[j]], add=True)
                if do_deg:
                    pltpu.sync_copy(ones_v, deg_sh.at[idx_v.at[j]], add=True)
        plsc.subcore_barrier()
        pltpu.sync_copy(acc_sh.at[pl.ds(r0, RPT)],
                        out_hbm.at[cid].at[pl.ds(r0, RPT)])
        if do_deg:
            pltpu.sync_copy(deg_sh.at[pl.ds(r0, RPT)],
                            deg_hbm.at[cid].at[pl.ds(r0, RPT)])

    if not do_deg:
        out_type = out_type[0]
    return pl.kernel(
        body, out_type=out_type, mesh=_sc_mesh(),
        compiler_params=pltpu.CompilerParams(use_tc_tiling_on_sc=False),
        scratch_types=scratch)


def _sc_scatter_deg(msg, idx3, zeros_np, ones_ch):
    return _make_sc_scatter(True)(msg, idx3, zeros_np, ones_ch)


def _sc_scatter(msg, idx3, zeros_np):
    return _make_sc_scatter(False)(msg, idx3, zeros_np)


def _msg_body(ea_ref, xs_ref, w1_ref, b1_ref, w2_ref, b2_ref, q_ref, s_ref,
              out_ref):
    f32 = jnp.float32
    ea40 = ea_ref[...]
    ea = jnp.concatenate(
        [ea40[:, e * 5:(e + 1) * 5] for e in range(8)], axis=0)
    xs8 = xs_ref[...]
    xs = jnp.concatenate(
        [xs8[:, e * FD:(e + 1) * FD] for e in range(8)], axis=0)
    h = jnp.maximum(
        jnp.dot(ea, w1_ref[...], preferred_element_type=f32)
        + b1_ref[...], 0.0)
    p = jnp.dot(h, w2_ref[...], preferred_element_type=f32) + b2_ref[...]
    xr = jnp.dot(xs, q_ref[...], preferred_element_type=f32)
    msg = jnp.dot(p * xr, s_ref[...], preferred_element_type=f32)
    out_ref[...] = jnp.concatenate(
        [msg[e * M8:(e + 1) * M8, :] for e in range(8)], axis=1)


def _msg_call(ea40, xs128, w1, b1r, w2, b2r, qmat, smat):
    c0 = lambda i: (0, 0)
    return pl.pallas_call(
        _msg_body,
        grid=(NEDGE // TE,),
        in_specs=[
            pl.BlockSpec((M8, 40), lambda i: (i, 0)),
            pl.BlockSpec((M8, 128), lambda i: (i, 0)),
            pl.BlockSpec((5, HMLP), c0),
            pl.BlockSpec((1, HMLP), c0),
            pl.BlockSpec((HMLP, FD * FD), c0),
            pl.BlockSpec((1, FD * FD), c0),
            pl.BlockSpec((FD, FD * FD), c0),
            pl.BlockSpec((FD * FD, FD), c0),
        ],
        out_specs=pl.BlockSpec((M8, 128), lambda i: (i, 0)),
        out_shape=jax.ShapeDtypeStruct((EPAD // 8, 128), jnp.float32),
    )(ea40, xs128, w1, b1r, w2, b2r, qmat, smat)


def _bn_relu(y, gamma, beta):
    m = jnp.mean(y, axis=0, keepdims=True)
    v = jnp.mean((y - m) * (y - m), axis=0, keepdims=True)
    return jnp.maximum((y - m) / jnp.sqrt(v + 1e-5) * gamma + beta, 0.0)


def _sum_parts(p):
    return p[0, :NNODE, :] + p[1, :NNODE, :]


def _combine1_body(p_ref, dp_ref, x_ref, root_ref, bias_ref, gamma_ref,
                   beta_ref, x2_ref, deg_ref):
    agg = _sum_parts(p_ref[...])
    deg = _sum_parts(dp_ref[...])
    y = (agg / jnp.maximum(deg, 1.0)
         + jnp.dot(x_ref[...], root_ref[...],
                   preferred_element_type=jnp.float32)
         + bias_ref[...])
    x2_ref[...] = _bn_relu(y, gamma_ref[...], beta_ref[...])
    deg_ref[...] = deg


def _combine1(parts, degp, x, root, bias, gamma, beta):
    return pl.pallas_call(
        _combine1_body,
        out_shape=[jax.ShapeDtypeStruct((NNODE, FD), jnp.float32),
                   jax.ShapeDtypeStruct((NNODE, FD), jnp.float32)],
    )(parts, degp, x, root, bias, gamma, beta)


def _combine2_body(p_ref, deg_ref, x_ref, root_ref, bias_ref, gamma_ref,
                   beta_ref, batch_ref, ga_ref, wh_ref, wg_ref, bfc_ref,
                   out_ref):
    f32 = jnp.float32
    agg = _sum_parts(p_ref[...])
    y = (agg / jnp.maximum(deg_ref[...], 1.0)
         + jnp.dot(x_ref[...], root_ref[...], preferred_element_type=f32)
         + bias_ref[...])
    h2 = _bn_relu(y, gamma_ref[...], beta_ref[...])
    gi = lax.broadcasted_iota(jnp.int32, (8, NNODE), 0)
    oh = (jnp.broadcast_to(batch_ref[...], (8, NNODE)) == gi).astype(f32)
    cnt = jnp.sum(oh, axis=1, keepdims=True)
    pooled = (jnp.dot(oh, h2, preferred_element_type=f32)
              / jnp.maximum(cnt, 1.0))
    out_ref[...] = (jnp.dot(pooled, wh_ref[...], preferred_element_type=f32)
                    + jnp.dot(ga_ref[...], wg_ref[...],
                              preferred_element_type=f32)
                    + bfc_ref[...])


def _combine2(parts, deg16, x2, root, bias, gamma, beta, batch2d, ga,
              wh, wg, bfc2d):
    return pl.pallas_call(
        _combine2_body,
        out_shape=jax.ShapeDtypeStruct((8, 128), jnp.float32),
    )(parts, deg16, x2, root, bias, gamma, beta, batch2d, ga, wh, wg, bfc2d)


def kernel(x, edge_index, edge_attr, batch, graph_attr, W1, b1, W2, b2,
           root1, bias1, gamma1, beta1, W3, b3, W4, b4, root2, bias2,
           gamma2, beta2, Wfc, bfc):
    f32 = jnp.float32
    src = edge_index[0].astype(jnp.int32)
    dst = edge_index[1].astype(jnp.int32)
    pad = EPAD - NEDGE
    src_p = jnp.concatenate(
        [src, jnp.zeros((pad + (NCHP - NCHT) * CH,), jnp.int32)]
    ).reshape(NCHP, CH)
    dst_p = jnp.concatenate(
        [dst, jnp.full((pad,), SENT, jnp.int32)]).reshape(NW, NCH, CH)
    ea40 = edge_attr.reshape(NEDGE // 8, 40)
    zeros_np = jnp.zeros((NPAD, FD), f32)
    ones_ch = jnp.ones((CH, FD), f32)

    qmat = jnp.repeat(jnp.eye(FD, dtype=f32), FD, axis=1)
    smat = jnp.tile(jnp.eye(FD, dtype=f32), (FD, 1))
    xs1 = _sc_gather(x, src_p)
    msg1 = _msg_call(ea40, xs1.reshape(EPAD // 8, 128), W1,
                     b1.reshape(1, HMLP), W2, b2.reshape(1, FD * FD),
                     qmat, smat)
    agg1, degp = _sc_scatter_deg(msg1.reshape(EPAD, FD), dst_p, zeros_np,
                                 ones_ch)
    x2, deg16 = _combine1(agg1, degp, x, root1, bias1.reshape(1, FD),
                          gamma1.reshape(1, FD), beta1.reshape(1, FD))

    xs2 = _sc_gather(x2, src_p)
    msg2 = _msg_call(ea40, xs2.reshape(EPAD // 8, 128), W3,
                     b3.reshape(1, HMLP), W4, b4.reshape(1, FD * FD),
                     qmat, smat)
    agg2 = _sc_scatter(msg2.reshape(EPAD, FD), dst_p, zeros_np)

    wh = jnp.zeros((FD, 128), f32).at[:, :3].set(Wfc[:FD])
    wg = jnp.zeros((16, 128), f32).at[:10, :3].set(Wfc[FD:])
    ga_p = jnp.zeros((8, 16), f32).at[:, :10].set(graph_attr)
    bfc2d = jnp.zeros((1, 128), f32).at[0, :3].set(bfc)
    out = _combine2(agg2, deg16, x2, root2, bias2.reshape(1, FD),
                    gamma2.reshape(1, FD), beta2.reshape(1, FD),
                    batch.astype(jnp.int32).reshape(1, NNODE), ga_p,
                    wh, wg, bfc2d)
    return out[:, :3]


# speedup vs baseline: 1.0943x; 1.0747x over previous
import functools

import jax
import jax.numpy as jnp
import numpy as np
from jax import lax
from jax.experimental import pallas as pl
from jax.experimental.pallas import tpu as pltpu
from jax.experimental.pallas import tpu_sc as plsc

NNODE = 10000
NEDGE = 160000
FD = 16
HMLP = 32

NC, NS = 2, 16
NW = NC * NS
CH = 128
GRP = 8
NCH = 40
PERW = NCH * CH
EPAD = NW * PERW
RPT = 626
NPAD = RPT * NS
SENT = NNODE
TE = 3200
M8 = TE // 8

_m = np.arange(EPAD)
_b, _l = _m // TE, _m % TE
EDGE_PERM = jnp.asarray(
    np.where(_m < NEDGE, _b * TE + (_l % 8) * M8 + _l // 8, _m),
    dtype=jnp.int32)


def _sc_mesh():
    return plsc.VectorSubcoreMesh(
        core_axis_name="c", subcore_axis_name="s",
        num_cores=NC, num_subcores=NS)


GN0 = 56
GN1 = 24
GNMX = max(GN0, GN1)
NCHT = EPAD // CH
NCHP = NCHT + GNMX - min(GN0, GN1)


@functools.cache
def _make_sc_gather():
    def body(x_hbm, idx_hbm, out_hbm, idx_v, rows_v, sem):
        cid = lax.axis_index("c")
        sid = lax.axis_index("s")
        start = jnp.where(cid == 0, sid * GN0, NS * GN0 + sid * GN1)
        ngrp = jnp.where(cid == 0, GN0 // GRP, GN1 // GRP)
        pltpu.sync_copy(idx_hbm.at[pl.ds(start, GNMX)], idx_v)
        for g in range(GNMX // GRP):
            @pl.when(g < ngrp)
            def _():
                cps = []
                for b in range(GRP):
                    j = g * GRP + b
                    cps.append(pltpu.async_copy(
                        x_hbm.at[idx_v.at[j]],
                        rows_v.at[pl.ds(b * CH, CH)], sem))
                for cp in cps:
                    cp.wait()
                pltpu.sync_copy(
                    rows_v,
                    out_hbm.at[pl.ds((start + g * GRP) * CH, GRP * CH)])

    return pl.kernel(
        body,
        out_type=jax.ShapeDtypeStruct((EPAD, FD), jnp.float32),
        mesh=_sc_mesh(),
        compiler_params=pltpu.CompilerParams(use_tc_tiling_on_sc=False),
        scratch_types=[
            pltpu.VMEM((GNMX, CH), jnp.int32),
            pltpu.VMEM((GRP * CH, FD), jnp.float32),
            pltpu.SemaphoreType.DMA,
        ])


def _sc_gather(x, idx2):
    return _make_sc_gather()(x, idx2)


@functools.cache
def _make_sc_scatter(do_deg):
    out_type = [jax.ShapeDtypeStruct((NC, NPAD, FD), jnp.float32)]
    scratch = [
        pltpu.VMEM((NCH, CH), jnp.int32),
        pltpu.VMEM((GRP * CH, FD), jnp.float32),
        pltpu.VMEM_SHARED((NPAD, FD), jnp.float32),
        pltpu.SemaphoreType.DMA,
    ]
    if do_deg:
        out_type.append(jax.ShapeDtypeStruct((NC, NPAD, FD), jnp.float32))
        scratch.append(pltpu.VMEM((CH, FD), jnp.float32))
        scratch.append(pltpu.VMEM_SHARED((NPAD, FD), jnp.float32))

    def body(*refs):
        if do_deg:
            (msg_hbm, idx_hbm, zeros_hbm, ones_hbm, out_hbm, deg_hbm,
             idx_v, msg_v, acc_sh, sem, ones_v, deg_sh) = refs
        else:
            (msg_hbm, idx_hbm, zeros_hbm, out_hbm,
             idx_v, msg_v, acc_sh, sem) = refs
        cid = lax.axis_index("c")
        sid = lax.axis_index("s")
        wid = sid * NC + cid
        r0 = sid * RPT
        pltpu.sync_copy(zeros_hbm.at[pl.ds(r0, RPT)],
                        acc_sh.at[pl.ds(r0, RPT)])
        if do_deg:
            pltpu.sync_copy(zeros_hbm.at[pl.ds(r0, RPT)],
                            deg_sh.at[pl.ds(r0, RPT)])
            pltpu.sync_copy(ones_hbm, ones_v)
        pltpu.sync_copy(idx_hbm.at[wid], idx_v)
        plsc.subcore_barrier()
        for g in range(NCH // GRP):
            pltpu.sync_copy(
                msg_hbm.at[pl.ds(wid * PERW + g * GRP * CH, GRP * CH)], msg_v)
            for b in range(GRP):
                j = g * GRP + b
                pltpu.sync_copy(msg_v.at[pl.ds(b * CH, CH)],
                                acc_sh.at[idx_v.at[j]], add=True)
                if do_deg:
                    pltpu.sync_copy(ones_v, deg_sh.at[idx_v.at[j]], add=True)
        plsc.subcore_barrier()
        pltpu.sync_copy(acc_sh.at[pl.ds(r0, RPT)],
                        out_hbm.at[cid].at[pl.ds(r0, RPT)])
        if do_deg:
            pltpu.sync_copy(deg_sh.at[pl.ds(r0, RPT)],
                            deg_hbm.at[cid].at[pl.ds(r0, RPT)])

    if not do_deg:
        out_type = out_type[0]
    return pl.kernel(
        body, out_type=out_type, mesh=_sc_mesh(),
        compiler_params=pltpu.CompilerParams(use_tc_tiling_on_sc=False),
        scratch_types=scratch)


def _sc_scatter_deg(msg, idx3, zeros_np, ones_ch):
    return _make_sc_scatter(True)(msg, idx3, zeros_np, ones_ch)


def _sc_scatter(msg, idx3, zeros_np):
    return _make_sc_scatter(False)(msg, idx3, zeros_np)


def _msg_body(ea_ref, xs_ref, w1_ref, b1_ref, w2_ref, b2_ref, q_ref, s_ref,
              out_ref):
    f32 = jnp.float32
    ea = ea_ref[...]
    xs8 = xs_ref[...]
    xs = jnp.concatenate(
        [xs8[:, e * FD:(e + 1) * FD] for e in range(8)], axis=0)
    h = jnp.maximum(
        jnp.dot(ea, w1_ref[...], preferred_element_type=f32)
        + b1_ref[...], 0.0)
    p = jnp.dot(h, w2_ref[...], preferred_element_type=f32) + b2_ref[...]
    xr = jnp.dot(xs, q_ref[...], preferred_element_type=f32)
    msg = jnp.dot(p * xr, s_ref[...], preferred_element_type=f32)
    out_ref[...] = jnp.concatenate(
        [msg[e * M8:(e + 1) * M8, :] for e in range(8)], axis=1)


def _msg_call(ea40, xs128, w1, b1r, w2, b2r, qmat, smat):
    c0 = lambda i: (0, 0)
    return pl.pallas_call(
        _msg_body,
        grid=(NEDGE // TE,),
        in_specs=[
            pl.BlockSpec((TE, 5), lambda i: (i, 0)),
            pl.BlockSpec((M8, 128), lambda i: (i, 0)),
            pl.BlockSpec((5, HMLP), c0),
            pl.BlockSpec((1, HMLP), c0),
            pl.BlockSpec((HMLP, FD * FD), c0),
            pl.BlockSpec((1, FD * FD), c0),
            pl.BlockSpec((FD, FD * FD), c0),
            pl.BlockSpec((FD * FD, FD), c0),
        ],
        out_specs=pl.BlockSpec((M8, 128), lambda i: (i, 0)),
        out_shape=jax.ShapeDtypeStruct((EPAD // 8, 128), jnp.float32),
    )(ea40, xs128, w1, b1r, w2, b2r, qmat, smat)


def _bn_relu(y, gamma, beta):
    m = jnp.mean(y, axis=0, keepdims=True)
    v = jnp.mean((y - m) * (y - m), axis=0, keepdims=True)
    return jnp.maximum((y - m) / jnp.sqrt(v + 1e-5) * gamma + beta, 0.0)


def _sum_parts(p):
    return p[0, :NNODE, :] + p[1, :NNODE, :]


def _combine1_body(p_ref, dp_ref, x_ref, root_ref, bias_ref, gamma_ref,
                   beta_ref, x2_ref, deg_ref):
    agg = _sum_parts(p_ref[...])
    deg = _sum_parts(dp_ref[...])
    y = (agg / jnp.maximum(deg, 1.0)
         + jnp.dot(x_ref[...], root_ref[...],
                   preferred_element_type=jnp.float32)
         + bias_ref[...])
    x2_ref[...] = _bn_relu(y, gamma_ref[...], beta_ref[...])
    deg_ref[...] = deg


def _combine1(parts, degp, x, root, bias, gamma, beta):
    return pl.pallas_call(
        _combine1_body,
        out_shape=[jax.ShapeDtypeStruct((NNODE, FD), jnp.float32),
                   jax.ShapeDtypeStruct((NNODE, FD), jnp.float32)],
    )(parts, degp, x, root, bias, gamma, beta)


def _combine2_body(p_ref, deg_ref, x_ref, root_ref, bias_ref, gamma_ref,
                   beta_ref, batch_ref, ga_ref, wh_ref, wg_ref, bfc_ref,
                   out_ref):
    f32 = jnp.float32
    agg = _sum_parts(p_ref[...])
    y = (agg / jnp.maximum(deg_ref[...], 1.0)
         + jnp.dot(x_ref[...], root_ref[...], preferred_element_type=f32)
         + bias_ref[...])
    h2 = _bn_relu(y, gamma_ref[...], beta_ref[...])
    gi = lax.broadcasted_iota(jnp.int32, (8, NNODE), 0)
    oh = (jnp.broadcast_to(batch_ref[...], (8, NNODE)) == gi).astype(f32)
    cnt = jnp.sum(oh, axis=1, keepdims=True)
    pooled = (jnp.dot(oh, h2, preferred_element_type=f32)
              / jnp.maximum(cnt, 1.0))
    out_ref[...] = (jnp.dot(pooled, wh_ref[...], preferred_element_type=f32)
                    + jnp.dot(ga_ref[...], wg_ref[...],
                              preferred_element_type=f32)
                    + bfc_ref[...])


def _combine2(parts, deg16, x2, root, bias, gamma, beta, batch2d, ga,
              wh, wg, bfc2d):
    return pl.pallas_call(
        _combine2_body,
        out_shape=jax.ShapeDtypeStruct((8, 128), jnp.float32),
    )(parts, deg16, x2, root, bias, gamma, beta, batch2d, ga, wh, wg, bfc2d)


def kernel(x, edge_index, edge_attr, batch, graph_attr, W1, b1, W2, b2,
           root1, bias1, gamma1, beta1, W3, b3, W4, b4, root2, bias2,
           gamma2, beta2, Wfc, bfc):
    f32 = jnp.float32
    src = edge_index[0].astype(jnp.int32)
    dst = edge_index[1].astype(jnp.int32)
    pad = EPAD - NEDGE
    src_p = jnp.concatenate(
        [jnp.concatenate([src, jnp.zeros((pad,), jnp.int32)])[EDGE_PERM],
         jnp.zeros(((NCHP - NCHT) * CH,), jnp.int32)]).reshape(NCHP, CH)
    dst_p = jnp.concatenate(
        [dst, jnp.full((pad,), SENT, jnp.int32)])[EDGE_PERM].reshape(
            NW, NCH, CH)
    zeros_np = jnp.zeros((NPAD, FD), f32)
    ones_ch = jnp.ones((CH, FD), f32)

    qmat = jnp.repeat(jnp.eye(FD, dtype=f32), FD, axis=1)
    smat = jnp.tile(jnp.eye(FD, dtype=f32), (FD, 1))
    xs1 = _sc_gather(x, src_p)
    msg1 = _msg_call(edge_attr, xs1.reshape(EPAD // 8, 128), W1,
                     b1.reshape(1, HMLP), W2, b2.reshape(1, FD * FD),
                     qmat, smat)
    agg1, degp = _sc_scatter_deg(msg1.reshape(EPAD, FD), dst_p, zeros_np,
                                 ones_ch)
    x2, deg16 = _combine1(agg1, degp, x, root1, bias1.reshape(1, FD),
                          gamma1.reshape(1, FD), beta1.reshape(1, FD))

    xs2 = _sc_gather(x2, src_p)
    msg2 = _msg_call(edge_attr, xs2.reshape(EPAD // 8, 128), W3,
                     b3.reshape(1, HMLP), W4, b4.reshape(1, FD * FD),
                     qmat, smat)
    agg2 = _sc_scatter(msg2.reshape(EPAD, FD), dst_p, zeros_np)

    wh = jnp.zeros((FD, 128), f32).at[:, :3].set(Wfc[:FD])
    wg = jnp.zeros((16, 128), f32).at[:10, :3].set(Wfc[FD:])
    ga_p = jnp.zeros((8, 16), f32).at[:, :10].set(graph_attr)
    bfc2d = jnp.zeros((1, 128), f32).at[0, :3].set(bfc)
    out = _combine2(agg2, deg16, x2, root2, bias2.reshape(1, FD),
                    gamma2.reshape(1, FD), beta2.reshape(1, FD),
                    batch.astype(jnp.int32).reshape(1, NNODE), ga_p,
                    wh, wg, bfc2d)
    return out[:, :3]
